# baseline (speedup 1.0000x reference)
import functools

import jax
import jax.numpy as jnp
from jax import lax
from jax.experimental import pallas as pl
from jax.experimental.pallas import tpu as pltpu
from jax.experimental.pallas import tpu_sc as plsc

B = 4
N = 16384
BN = B * N
C = 96
LVLS = ((224, 224), (112, 112), (56, 56))
NC, NS, L = 2, 16, 16
NW = NC * NS
PW = BN // NW
P = 32
NCHUNK = PW // P
NG = P // L
DL = 3 * C
CP = 128
DOUT = 256


def _sc_interp(f0, f1, f2, xs, ys):
    mesh = plsc.VectorSubcoreMesh(core_axis_name="c", subcore_axis_name="s")

    scratch = (
        [pltpu.VMEM((PW,), jnp.float32)] * 2
        + [pltpu.VMEM((P,), jnp.int32)] * 24
        + [pltpu.VMEM((P,), jnp.float32)] * 24
        + [pltpu.VMEM((P, CP), jnp.float32)] * 24
        + [pltpu.VMEM((P, DL), jnp.float32)]
        + [pltpu.SemaphoreType.DMA] * 2
    )

    @functools.partial(
        pl.kernel,
        out_type=jax.ShapeDtypeStruct((BN, DL), jnp.float32),
        mesh=mesh,
        scratch_types=scratch,
    )
    def k(f0_h, f1_h, f2_h, xs_h, ys_h, lat_h, *s):
        xs_v, ys_v = s[0], s[1]
        idx_v = [[s[2 + 12 * t + i] for i in range(12)] for t in range(2)]
        w_v = [[s[26 + 12 * t + i] for i in range(12)] for t in range(2)]
        rows_v = [[s[50 + 12 * t + i] for i in range(12)] for t in range(2)]
        lat_v = s[74]
        sems = (s[75], s[76])
        feats = (f0_h, f1_h, f2_h)

        wid = lax.axis_index("s") * NC + lax.axis_index("c")
        base = wid * PW
        b = wid // (N // PW)
        pltpu.sync_copy(xs_h.at[pl.ds(base, PW)], xs_v)
        pltpu.sync_copy(ys_h.at[pl.ds(base, PW)], ys_v)

        def idx_comp(ci, t):
            off = ci * P
            for l, (H, W) in enumerate(LVLS):
                boff = b * (H * W)
                for g in range(NG):
                    sl = pl.ds(off + g * L, L)
                    gsl = pl.ds(g * L, L)
                    x = xs_v[sl] * float(W - 1)
                    y = ys_v[sl] * float(H - 1)
                    xi = jnp.minimum(x.astype(jnp.int32), W - 2)
                    yi = jnp.minimum(y.astype(jnp.int32), H - 2)
                    wx = x - xi.astype(jnp.float32)
                    wy = y - yi.astype(jnp.float32)
                    r00 = yi * W + xi + boff
                    idx_v[t][4 * l][gsl] = r00
                    idx_v[t][4 * l + 1][gsl] = r00 + 1
                    idx_v[t][4 * l + 2][gsl] = r00 + W
                    idx_v[t][4 * l + 3][gsl] = r00 + W + 1
                    w_v[t][4 * l][gsl] = (1.0 - wx) * (1.0 - wy)
                    w_v[t][4 * l + 1][gsl] = wx * (1.0 - wy)
                    w_v[t][4 * l + 2][gsl] = (1.0 - wx) * wy
                    w_v[t][4 * l + 3][gsl] = wx * wy

        def fire(t):
            for l in range(3):
                for cnr in range(4):
                    pltpu.async_copy(feats[l].at[idx_v[t][4 * l + cnr]],
                                     rows_v[t][4 * l + cnr], sems[t])

        def drain(t):
            for l in range(3):
                for cnr in range(4):
                    pltpu.make_async_copy(feats[l].at[idx_v[t][4 * l + cnr]],
                                          rows_v[t][4 * l + cnr],
                                          sems[t]).wait()

        def combine_store(ci, t):
            for l in range(3):
                col0 = l * C
                r00v, r01v, r10v, r11v = rows_v[t][4 * l:4 * l + 4]
                w00v, w01v, w10v, w11v = w_v[t][4 * l:4 * l + 4]

                def grp_body(g, c2, r00v=r00v, r01v=r01v, r10v=r10v,
                             r11v=r11v, w00v=w00v, w01v=w01v, w10v=w10v,
                             w11v=w11v, col0=col0):
                    gsl = pl.ds(g * L, L)
                    w00g = w00v[gsl]
                    w01g = w01v[gsl]
                    w10g = w10v[gsl]
                    w11g = w11v[gsl]
                    for lane in range(L):
                        p = g * L + lane
                        w00 = jnp.broadcast_to(w00g[lane], (L,))
                        w01 = jnp.broadcast_to(w01g[lane], (L,))
                        w10 = jnp.broadcast_to(w10g[lane], (L,))
                        w11 = jnp.broadcast_to(w11g[lane], (L,))
                        for kk in range(C // L):
                            csl = pl.ds(kk * L, L)
                            acc = (w00 * r00v[p, csl] + w01 * r01v[p, csl]
                                   + w10 * r10v[p, csl] + w11 * r11v[p, csl])
                            lat_v[p, pl.ds(col0 + kk * L, L)] = acc
                    return c2
                lax.fori_loop(0, NG, grp_body, 0)
            pltpu.sync_copy(lat_v, lat_h.at[pl.ds(base + ci * P, P), :])

        idx_comp(0, 0)
        fire(0)

        def pair_body(ci2, carry):
            e = 2 * ci2
            idx_comp(e + 1, 1)
            fire(1)
            drain(0)
            combine_store(e, 0)

            @pl.when(e + 2 < NCHUNK)
            def _():
                idx_comp(e + 2, 0)
                fire(0)
            drain(1)
            combine_store(e + 1, 1)
            return carry

        lax.fori_loop(0, NCHUNK // 2, pair_body, 0)

    return k(f0, f1, f2, xs, ys)


def _tc_pad(f, bh):
    _, H, W, _ = f.shape
    br = bh * W

    def body(x_ref, o_ref):
        o_ref[:, :C] = x_ref[...].reshape(br, C)
        o_ref[:, C:] = jnp.zeros((br, CP - C), jnp.float32)

    return pl.pallas_call(
        body,
        grid=(B, H // bh),
        in_specs=[pl.BlockSpec((1, bh, W, C), lambda i, j: (i, j, 0, 0))],
        out_specs=pl.BlockSpec((br, CP), lambda i, j: (i * (H // bh) + j, 0)),
        out_shape=jax.ShapeDtypeStruct((B * H * W, CP), jnp.float32),
    )(f)


def _tc_mix(lat, pf, w, wp, bias):
    BM = 2048

    def body(lat_ref, p_ref, w_ref, wp_ref, b_ref, o_ref):
        acc = jnp.dot(lat_ref[...], w_ref[...],
                      preferred_element_type=jnp.float32)
        acc += p_ref[:, 0:1] * wp_ref[0:1, :]
        acc += p_ref[:, 1:2] * wp_ref[1:2, :]
        o_ref[...] = jnp.maximum(acc + b_ref[...], 0.0)

    return pl.pallas_call(
        body,
        grid=(BN // BM,),
        in_specs=[
            pl.BlockSpec((BM, DL), lambda i: (i, 0)),
            pl.BlockSpec((BM, 2), lambda i: (i, 0)),
            pl.BlockSpec((DL, DOUT), lambda i: (0, 0)),
            pl.BlockSpec((2, DOUT), lambda i: (0, 0)),
            pl.BlockSpec((1, DOUT), lambda i: (0, 0)),
        ],
        out_specs=pl.BlockSpec((BM, DOUT), lambda i: (i, 0)),
        out_shape=jax.ShapeDtypeStruct((BN, DOUT), jnp.float32),
    )(lat, pf, w, wp, bias.reshape(1, DOUT))


def kernel(feat0, feat1, feat2, pos, W_mix, b_mix):
    f0 = _tc_pad(feat0, 32)
    f1 = _tc_pad(feat1, 16)
    f2 = _tc_pad(feat2, 8)
    pf = pos.reshape(BN, 2)
    lat = _sc_interp(f0, f1, f2, pf[:, 0], pf[:, 1])
    return _tc_mix(lat, pf, W_mix[2:], W_mix[:2], b_mix)


# speedup vs baseline: 1.3807x; 1.3807x over previous
import functools

import jax
import jax.numpy as jnp
from jax import lax
from jax.experimental import pallas as pl
from jax.experimental.pallas import tpu as pltpu
from jax.experimental.pallas import tpu_sc as plsc

B = 4
N = 16384
BN = B * N
C = 96
LVLS = ((224, 224), (112, 112), (56, 56))
NC, NS, L = 2, 16, 16
NW = NC * NS
PW = BN // NW
P = 64
NCHUNK = PW // P
NG = P // L
DL = 3 * C
CP = 128
DOUT = 256


def _sc_interp(f0, f1, f2, xs, ys):
    mesh = plsc.VectorSubcoreMesh(core_axis_name="c", subcore_axis_name="s")

    scratch = (
        [pltpu.VMEM((PW,), jnp.float32)] * 2
        + [pltpu.VMEM((P,), jnp.int32)] * 12
        + [pltpu.VMEM((P,), jnp.float32)] * 12
        + [pltpu.VMEM((P, CP), jnp.float32)] * 12
        + [pltpu.VMEM((P, DL), jnp.float32)]
        + [pltpu.SemaphoreType.DMA]
    )

    @functools.partial(
        pl.kernel,
        out_type=jax.ShapeDtypeStruct((BN, DL), jnp.float32),
        mesh=mesh,
        scratch_types=scratch,
        compiler_params=pltpu.CompilerParams(needs_layout_passes=False),
    )
    def k(f0_h, f1_h, f2_h, xs_h, ys_h, lat_h, *s):
        xs_v, ys_v = s[0], s[1]
        idx_v = [s[2 + i] for i in range(12)]
        w_v = [s[14 + i] for i in range(12)]
        rows_v = [s[26 + i] for i in range(12)]
        lat_v = s[38]
        sem = s[39]
        feats = (f0_h, f1_h, f2_h)

        wid = lax.axis_index("s") * NC + lax.axis_index("c")
        base = wid * PW
        b = wid // (N // PW)
        pltpu.sync_copy(xs_h.at[pl.ds(base, PW)], xs_v)
        pltpu.sync_copy(ys_h.at[pl.ds(base, PW)], ys_v)

        def chunk_body(ci, carry):
            off = ci * P
            for l, (H, W) in enumerate(LVLS):
                boff = b * (H * W)
                for g in range(NG):
                    sl = pl.ds(off + g * L, L)
                    gsl = pl.ds(g * L, L)
                    x = xs_v[sl] * float(W - 1)
                    y = ys_v[sl] * float(H - 1)
                    xi = jnp.minimum(x.astype(jnp.int32), W - 2)
                    yi = jnp.minimum(y.astype(jnp.int32), H - 2)
                    wx = x - xi.astype(jnp.float32)
                    wy = y - yi.astype(jnp.float32)
                    r00 = yi * W + xi + boff
                    idx_v[4 * l][gsl] = r00
                    idx_v[4 * l + 1][gsl] = r00 + 1
                    idx_v[4 * l + 2][gsl] = r00 + W
                    idx_v[4 * l + 3][gsl] = r00 + W + 1
                    w_v[4 * l][gsl] = (1.0 - wx) * (1.0 - wy)
                    w_v[4 * l + 1][gsl] = wx * (1.0 - wy)
                    w_v[4 * l + 2][gsl] = (1.0 - wx) * wy
                    w_v[4 * l + 3][gsl] = wx * wy
            copies = []
            for l in range(3):
                for cnr in range(4):
                    copies.append(pltpu.async_copy(
                        feats[l].at[idx_v[4 * l + cnr]],
                        rows_v[4 * l + cnr], sem))
            for cp in copies:
                cp.wait()
            for l in range(3):
                col0 = l * C
                r00v, r01v, r10v, r11v = rows_v[4 * l:4 * l + 4]
                w00v, w01v, w10v, w11v = w_v[4 * l:4 * l + 4]

                @plsc.parallel_loop(0, P, 1, unroll=4)
                def pt_body(p, r00v=r00v, r01v=r01v, r10v=r10v,
                            r11v=r11v, w00v=w00v, w01v=w01v, w10v=w10v,
                            w11v=w11v, col0=col0):
                    pv = jnp.full((L,), p, jnp.int32)
                    w00 = plsc.load_gather(w00v, [pv])
                    w01 = plsc.load_gather(w01v, [pv])
                    w10 = plsc.load_gather(w10v, [pv])
                    w11 = plsc.load_gather(w11v, [pv])
                    for kk in range(C // L):
                        csl = pl.ds(kk * L, L)
                        acc = (w00 * r00v[p, csl] + w01 * r01v[p, csl]
                               + w10 * r10v[p, csl] + w11 * r11v[p, csl])
                        lat_v[p, pl.ds(col0 + kk * L, L)] = acc
            pltpu.sync_copy(lat_v, lat_h.at[pl.ds(base + off, P), :])
            return carry

        lax.fori_loop(0, NCHUNK, chunk_body, 0)

    return k(f0, f1, f2, xs, ys)


def _tc_pad(f, bh):
    _, H, W, _ = f.shape
    br = bh * W

    def body(x_ref, o_ref):
        o_ref[:, :C] = x_ref[...].reshape(br, C)
        o_ref[:, C:] = jnp.zeros((br, CP - C), jnp.float32)

    return pl.pallas_call(
        body,
        grid=(B, H // bh),
        in_specs=[pl.BlockSpec((1, bh, W, C), lambda i, j: (i, j, 0, 0))],
        out_specs=pl.BlockSpec((br, CP), lambda i, j: (i * (H // bh) + j, 0)),
        out_shape=jax.ShapeDtypeStruct((B * H * W, CP), jnp.float32),
    )(f)


def _tc_mix(lat, pf, w, wp, bias):
    BM = 2048

    def body(lat_ref, p_ref, w_ref, wp_ref, b_ref, o_ref):
        acc = jnp.dot(lat_ref[...], w_ref[...],
                      preferred_element_type=jnp.float32)
        acc += p_ref[:, 0:1] * wp_ref[0:1, :]
        acc += p_ref[:, 1:2] * wp_ref[1:2, :]
        o_ref[...] = jnp.maximum(acc + b_ref[...], 0.0)

    return pl.pallas_call(
        body,
        grid=(BN // BM,),
        in_specs=[
            pl.BlockSpec((BM, DL), lambda i: (i, 0)),
            pl.BlockSpec((BM, 2), lambda i: (i, 0)),
            pl.BlockSpec((DL, DOUT), lambda i: (0, 0)),
            pl.BlockSpec((2, DOUT), lambda i: (0, 0)),
            pl.BlockSpec((1, DOUT), lambda i: (0, 0)),
        ],
        out_specs=pl.BlockSpec((BM, DOUT), lambda i: (i, 0)),
        out_shape=jax.ShapeDtypeStruct((BN, DOUT), jnp.float32),
    )(lat, pf, w, wp, bias.reshape(1, DOUT))


def kernel(feat0, feat1, feat2, pos, W_mix, b_mix):
    f0 = _tc_pad(feat0, 32)
    f1 = _tc_pad(feat1, 16)
    f2 = _tc_pad(feat2, 8)
    pf = pos.reshape(BN, 2)
    lat = _sc_interp(f0, f1, f2, pf[:, 0], pf[:, 1])
    return _tc_mix(lat, pf, W_mix[2:], W_mix[:2], b_mix)


# speedup vs baseline: 1.5824x; 1.1461x over previous
import functools

import jax
import jax.numpy as jnp
from jax import lax
from jax.experimental import pallas as pl
from jax.experimental.pallas import tpu as pltpu
from jax.experimental.pallas import tpu_sc as plsc

B = 4
N = 16384
BN = B * N
C = 96
LVLS = ((224, 224), (112, 112), (56, 56))
NC, NS, L = 2, 16, 16
NW = NC * NS
PW = BN // NW
P = 32
NCHUNK = PW // P
NG = P // L
DL = 3 * C
CP = 128
DOUT = 256


def _sc_interp(f0, f1, f2, xs, ys):
    mesh = plsc.VectorSubcoreMesh(core_axis_name="c", subcore_axis_name="s")

    scratch = (
        [pltpu.VMEM((PW,), jnp.float32)] * 2
        + [pltpu.VMEM((P,), jnp.int32)] * 24
        + [pltpu.VMEM((P,), jnp.float32)] * 24
        + [pltpu.VMEM((P, CP), jnp.float32)] * 24
        + [pltpu.VMEM((P, DL), jnp.float32)]
        + [pltpu.SemaphoreType.DMA] * 2
    )

    @functools.partial(
        pl.kernel,
        out_type=jax.ShapeDtypeStruct((BN, DL), jnp.float32),
        mesh=mesh,
        scratch_types=scratch,
        compiler_params=pltpu.CompilerParams(needs_layout_passes=False),
    )
    def k(f0_h, f1_h, f2_h, xs_h, ys_h, lat_h, *s):
        xs_v, ys_v = s[0], s[1]
        idx_v = [[s[2 + 12 * t + i] for i in range(12)] for t in range(2)]
        w_v = [[s[26 + 12 * t + i] for i in range(12)] for t in range(2)]
        rows_v = [[s[50 + 12 * t + i] for i in range(12)] for t in range(2)]
        lat_v = s[74]
        sems = (s[75], s[76])
        feats = (f0_h, f1_h, f2_h)

        wid = lax.axis_index("s") * NC + lax.axis_index("c")
        base = wid * PW
        b = wid // (N // PW)
        pltpu.sync_copy(xs_h.at[pl.ds(base, PW)], xs_v)
        pltpu.sync_copy(ys_h.at[pl.ds(base, PW)], ys_v)

        def idx_comp(ci, t):
            off = ci * P
            for l, (H, W) in enumerate(LVLS):
                boff = b * (H * W)
                for g in range(NG):
                    sl = pl.ds(off + g * L, L)
                    gsl = pl.ds(g * L, L)
                    x = xs_v[sl] * float(W - 1)
                    y = ys_v[sl] * float(H - 1)
                    xi = jnp.minimum(x.astype(jnp.int32), W - 2)
                    yi = jnp.minimum(y.astype(jnp.int32), H - 2)
                    wx = x - xi.astype(jnp.float32)
                    wy = y - yi.astype(jnp.float32)
                    r00 = yi * W + xi + boff
                    idx_v[t][4 * l][gsl] = r00
                    idx_v[t][4 * l + 1][gsl] = r00 + 1
                    idx_v[t][4 * l + 2][gsl] = r00 + W
                    idx_v[t][4 * l + 3][gsl] = r00 + W + 1
                    w_v[t][4 * l][gsl] = (1.0 - wx) * (1.0 - wy)
                    w_v[t][4 * l + 1][gsl] = wx * (1.0 - wy)
                    w_v[t][4 * l + 2][gsl] = (1.0 - wx) * wy
                    w_v[t][4 * l + 3][gsl] = wx * wy

        def fire(t):
            for l in range(3):
                for cnr in range(4):
                    pltpu.async_copy(feats[l].at[idx_v[t][4 * l + cnr]],
                                     rows_v[t][4 * l + cnr], sems[t])

        def drain(t):
            for l in range(3):
                for cnr in range(4):
                    pltpu.make_async_copy(feats[l].at[idx_v[t][4 * l + cnr]],
                                          rows_v[t][4 * l + cnr],
                                          sems[t]).wait()

        def combine_store(ci, t):
            for l in range(3):
                col0 = l * C
                r00v, r01v, r10v, r11v = rows_v[t][4 * l:4 * l + 4]
                w00v, w01v, w10v, w11v = w_v[t][4 * l:4 * l + 4]

                @plsc.parallel_loop(0, P, 1, unroll=4)
                def pt_body(p, r00v=r00v, r01v=r01v, r10v=r10v,
                            r11v=r11v, w00v=w00v, w01v=w01v, w10v=w10v,
                            w11v=w11v, col0=col0):
                    pv = jnp.full((L,), p, jnp.int32)
                    w00 = plsc.load_gather(w00v, [pv])
                    w01 = plsc.load_gather(w01v, [pv])
                    w10 = plsc.load_gather(w10v, [pv])
                    w11 = plsc.load_gather(w11v, [pv])
                    for kk in range(C // L):
                        csl = pl.ds(kk * L, L)
                        acc = (w00 * r00v[p, csl] + w01 * r01v[p, csl]
                               + w10 * r10v[p, csl] + w11 * r11v[p, csl])
                        lat_v[p, pl.ds(col0 + kk * L, L)] = acc
            pltpu.sync_copy(lat_v, lat_h.at[pl.ds(base + ci * P, P), :])

        idx_comp(0, 0)
        fire(0)

        def pair_body(ci2, carry):
            e = 2 * ci2
            idx_comp(e + 1, 1)
            fire(1)
            drain(0)
            combine_store(e, 0)

            @pl.when(e + 2 < NCHUNK)
            def _():
                idx_comp(e + 2, 0)
                fire(0)
            drain(1)
            combine_store(e + 1, 1)
            return carry

        lax.fori_loop(0, NCHUNK // 2, pair_body, 0)

    return k(f0, f1, f2, xs, ys)


def _tc_pad(f, bh):
    _, H, W, _ = f.shape
    br = bh * W

    def body(x_ref, o_ref):
        o_ref[:, :C] = x_ref[...].reshape(br, C)
        o_ref[:, C:] = jnp.zeros((br, CP - C), jnp.float32)

    return pl.pallas_call(
        body,
        grid=(B, H // bh),
        in_specs=[pl.BlockSpec((1, bh, W, C), lambda i, j: (i, j, 0, 0))],
        out_specs=pl.BlockSpec((br, CP), lambda i, j: (i * (H // bh) + j, 0)),
        out_shape=jax.ShapeDtypeStruct((B * H * W, CP), jnp.float32),
    )(f)


def _tc_mix(lat, pf, w, wp, bias):
    BM = 2048

    def body(lat_ref, p_ref, w_ref, wp_ref, b_ref, o_ref):
        acc = jnp.dot(lat_ref[...], w_ref[...],
                      preferred_element_type=jnp.float32)
        acc += p_ref[:, 0:1] * wp_ref[0:1, :]
        acc += p_ref[:, 1:2] * wp_ref[1:2, :]
        o_ref[...] = jnp.maximum(acc + b_ref[...], 0.0)

    return pl.pallas_call(
        body,
        grid=(BN // BM,),
        in_specs=[
            pl.BlockSpec((BM, DL), lambda i: (i, 0)),
            pl.BlockSpec((BM, 2), lambda i: (i, 0)),
            pl.BlockSpec((DL, DOUT), lambda i: (0, 0)),
            pl.BlockSpec((2, DOUT), lambda i: (0, 0)),
            pl.BlockSpec((1, DOUT), lambda i: (0, 0)),
        ],
        out_specs=pl.BlockSpec((BM, DOUT), lambda i: (i, 0)),
        out_shape=jax.ShapeDtypeStruct((BN, DOUT), jnp.float32),
    )(lat, pf, w, wp, bias.reshape(1, DOUT))


def kernel(feat0, feat1, feat2, pos, W_mix, b_mix):
    f0 = _tc_pad(feat0, 32)
    f1 = _tc_pad(feat1, 16)
    f2 = _tc_pad(feat2, 8)
    pf = pos.reshape(BN, 2)
    lat = _sc_interp(f0, f1, f2, pf[:, 0], pf[:, 1])
    return _tc_mix(lat, pf, W_mix[2:], W_mix[:2], b_mix)


# speedup vs baseline: 1.6149x; 1.0205x over previous
import functools

import jax
import jax.numpy as jnp
from jax import lax
from jax.experimental import pallas as pl
from jax.experimental.pallas import tpu as pltpu
from jax.experimental.pallas import tpu_sc as plsc

B = 4
N = 16384
BN = B * N
C = 96
LVLS = ((224, 224), (112, 112), (56, 56))
NC, NS, L = 2, 16, 16
NW = NC * NS
PW = BN // NW
P = 32
NCHUNK = PW // P
NG = P // L
DL = 3 * C
CP = 128
DOUT = 256


def _sc_interp(f0, f1, f2, xs, ys):
    mesh = plsc.VectorSubcoreMesh(core_axis_name="c", subcore_axis_name="s")

    scratch = (
        [pltpu.VMEM((PW,), jnp.float32)] * 2
        + [pltpu.VMEM((P,), jnp.int32)] * 24
        + [pltpu.VMEM((P,), jnp.float32)] * 24
        + [pltpu.VMEM((P, CP), jnp.float32)] * 24
        + [pltpu.VMEM((P, DL), jnp.float32)]
        + [pltpu.SemaphoreType.DMA] * 2
    )

    @functools.partial(
        pl.kernel,
        out_type=jax.ShapeDtypeStruct((BN, DL), jnp.float32),
        mesh=mesh,
        scratch_types=scratch,
        compiler_params=pltpu.CompilerParams(needs_layout_passes=False),
    )
    def k(f0_h, f1_h, f2_h, xs_h, ys_h, lat_h, *s):
        xs_v, ys_v = s[0], s[1]
        idx_v = [[s[2 + 12 * t + i] for i in range(12)] for t in range(2)]
        w_v = [[s[26 + 12 * t + i] for i in range(12)] for t in range(2)]
        rows_v = [[s[50 + 12 * t + i] for i in range(12)] for t in range(2)]
        lat_v = s[74]
        sems = (s[75], s[76])
        feats = (f0_h, f1_h, f2_h)

        wid = lax.axis_index("s") * NC + lax.axis_index("c")
        base = wid * PW
        b = wid // (N // PW)
        pltpu.sync_copy(xs_h.at[pl.ds(base, PW)], xs_v)
        pltpu.sync_copy(ys_h.at[pl.ds(base, PW)], ys_v)

        def idx_comp(ci, t):
            off = ci * P
            for l, (H, W) in enumerate(LVLS):
                boff = b * (H * W)
                for g in range(NG):
                    sl = pl.ds(off + g * L, L)
                    gsl = pl.ds(g * L, L)
                    x = xs_v[sl] * float(W - 1)
                    y = ys_v[sl] * float(H - 1)
                    xi = jnp.minimum(x.astype(jnp.int32), W - 2)
                    yi = jnp.minimum(y.astype(jnp.int32), H - 2)
                    wx = x - xi.astype(jnp.float32)
                    wy = y - yi.astype(jnp.float32)
                    r00 = yi * W + xi + boff
                    idx_v[t][4 * l][gsl] = r00
                    idx_v[t][4 * l + 1][gsl] = r00 + 1
                    idx_v[t][4 * l + 2][gsl] = r00 + W
                    idx_v[t][4 * l + 3][gsl] = r00 + W + 1
                    w_v[t][4 * l][gsl] = (1.0 - wx) * (1.0 - wy)
                    w_v[t][4 * l + 1][gsl] = wx * (1.0 - wy)
                    w_v[t][4 * l + 2][gsl] = (1.0 - wx) * wy
                    w_v[t][4 * l + 3][gsl] = wx * wy

        def fire(t):
            for l in range(3):
                for cnr in range(4):
                    pltpu.async_copy(feats[l].at[idx_v[t][4 * l + cnr]],
                                     rows_v[t][4 * l + cnr], sems[t])

        def drain(t):
            for l in range(3):
                for cnr in range(4):
                    pltpu.make_async_copy(feats[l].at[idx_v[t][4 * l + cnr]],
                                          rows_v[t][4 * l + cnr],
                                          sems[t]).wait()

        def combine_store(ci, t):
            for l in range(3):
                col0 = l * C
                r00v, r01v, r10v, r11v = rows_v[t][4 * l:4 * l + 4]
                w00v, w01v, w10v, w11v = w_v[t][4 * l:4 * l + 4]

                @plsc.parallel_loop(0, P, 1, unroll=8)
                def pt_body(p, r00v=r00v, r01v=r01v, r10v=r10v,
                            r11v=r11v, w00v=w00v, w01v=w01v, w10v=w10v,
                            w11v=w11v, col0=col0):
                    pv = jnp.full((L,), p, jnp.int32)
                    w00 = plsc.load_gather(w00v, [pv])
                    w01 = plsc.load_gather(w01v, [pv])
                    w10 = plsc.load_gather(w10v, [pv])
                    w11 = plsc.load_gather(w11v, [pv])
                    for kk in range(C // L):
                        csl = pl.ds(kk * L, L)
                        acc = (w00 * r00v[p, csl] + w01 * r01v[p, csl]
                               + w10 * r10v[p, csl] + w11 * r11v[p, csl])
                        lat_v[p, pl.ds(col0 + kk * L, L)] = acc
            pltpu.sync_copy(lat_v, lat_h.at[pl.ds(base + ci * P, P), :])

        idx_comp(0, 0)
        fire(0)

        def pair_body(ci2, carry):
            e = 2 * ci2
            idx_comp(e + 1, 1)
            fire(1)
            drain(0)
            combine_store(e, 0)

            @pl.when(e + 2 < NCHUNK)
            def _():
                idx_comp(e + 2, 0)
                fire(0)
            drain(1)
            combine_store(e + 1, 1)
            return carry

        lax.fori_loop(0, NCHUNK // 2, pair_body, 0)

    return k(f0, f1, f2, xs, ys)


def _tc_pad(f, bh):
    _, H, W, _ = f.shape
    br = bh * W

    def body(x_ref, o_ref):
        o_ref[:, :C] = x_ref[...].reshape(br, C)
        o_ref[:, C:] = jnp.zeros((br, CP - C), jnp.float32)

    return pl.pallas_call(
        body,
        grid=(B, H // bh),
        in_specs=[pl.BlockSpec((1, bh, W, C), lambda i, j: (i, j, 0, 0))],
        out_specs=pl.BlockSpec((br, CP), lambda i, j: (i * (H // bh) + j, 0)),
        out_shape=jax.ShapeDtypeStruct((B * H * W, CP), jnp.float32),
    )(f)


def _tc_mix(lat, pf, w, wp, bias):
    BM = 2048

    def body(lat_ref, p_ref, w_ref, wp_ref, b_ref, o_ref):
        acc = jnp.dot(lat_ref[...], w_ref[...],
                      preferred_element_type=jnp.float32)
        acc += p_ref[:, 0:1] * wp_ref[0:1, :]
        acc += p_ref[:, 1:2] * wp_ref[1:2, :]
        o_ref[...] = jnp.maximum(acc + b_ref[...], 0.0)

    return pl.pallas_call(
        body,
        grid=(BN // BM,),
        in_specs=[
            pl.BlockSpec((BM, DL), lambda i: (i, 0)),
            pl.BlockSpec((BM, 2), lambda i: (i, 0)),
            pl.BlockSpec((DL, DOUT), lambda i: (0, 0)),
            pl.BlockSpec((2, DOUT), lambda i: (0, 0)),
            pl.BlockSpec((1, DOUT), lambda i: (0, 0)),
        ],
        out_specs=pl.BlockSpec((BM, DOUT), lambda i: (i, 0)),
        out_shape=jax.ShapeDtypeStruct((BN, DOUT), jnp.float32),
    )(lat, pf, w, wp, bias.reshape(1, DOUT))


def kernel(feat0, feat1, feat2, pos, W_mix, b_mix):
    f0 = _tc_pad(feat0, 32)
    f1 = _tc_pad(feat1, 16)
    f2 = _tc_pad(feat2, 8)
    pf = pos.reshape(BN, 2)
    lat = _sc_interp(f0, f1, f2, pf[:, 0], pf[:, 1])
    return _tc_mix(lat, pf, W_mix[2:], W_mix[:2], b_mix)
